# baseline (speedup 1.0000x reference)
import functools

import jax
import jax.numpy as jnp
from jax.experimental import pallas as pl

A = 100000
B = 8
C = 20
M = 64
MARGIN = 8.0

BLOCK_A = 8192


def _loss_kernel(cls_ref, reg_ref, anc_ref, ann_ref, annq_ref, sums_ref):
    i = pl.program_id(1)

    @pl.when(i == 0)
    def _init():
        sums_ref[...] = jnp.zeros_like(sums_ref)

    x = cls_ref[0]
    r = reg_ref[0]
    ax0 = anc_ref[0:1, :]
    ay0 = anc_ref[1:2, :]
    ax1 = anc_ref[2:3, :]
    ay1 = anc_ref[3:4, :]
    q = ann_ref[0]

    bx0 = q[:, 0:1]
    by0 = q[:, 1:2]
    bx1 = q[:, 2:3]
    by1 = q[:, 3:4]
    blab = q[:, 4:5]
    bsqa = q[:, 5:6]
    barea = q[:, 6:7]
    bvalid = q[:, 7:8]

    area_a = (ax1 - ax0) * (ay1 - ay0)
    iw = jnp.maximum(jnp.minimum(ax1, bx1) - jnp.maximum(ax0, bx0), 0.0)
    ih = jnp.maximum(jnp.minimum(ay1, by1) - jnp.maximum(ay0, by0), 0.0)
    inter = iw * ih
    union = area_a + barea - inter
    iou = inter / jnp.maximum(union, 1e-8)
    iou = jnp.where(bvalid > 0.0, iou, -1.0)

    best = jnp.max(iou, axis=0, keepdims=True)
    m_iota = jax.lax.broadcasted_iota(jnp.int32, (M, 1), 0)
    argsel = jnp.min(jnp.where(iou == best, m_iota, M), axis=0, keepdims=True)
    onehot = (m_iota == argsel).astype(jnp.float32)

    qT = annq_ref[0]
    gall = jax.lax.dot_general(
        qT, onehot, (((1,), (0,)), ((), ())),
        preferred_element_type=jnp.float32)
    gx0 = gall[0:1, :]
    gy0 = gall[1:2, :]
    gx1 = gall[2:3, :]
    gy1 = gall[3:4, :]
    glab = gall[4:5, :]
    gsqa = gall[5:6, :]

    lane = jax.lax.broadcasted_iota(jnp.int32, (1, BLOCK_A), 1)
    in_range = (i * BLOCK_A + lane) < A
    is_pos = jnp.logical_and(best >= 0.5, in_range)
    keep = jnp.logical_and(jnp.logical_or(is_pos, best < 0.4), in_range)
    posf = jnp.where(is_pos, 1.0, 0.0)
    keepf = jnp.where(keep, 1.0, 0.0)

    t_cls = jnp.where(is_pos, (glab + 0.5).astype(jnp.int32), 0)
    margin = jnp.where(is_pos, MARGIN / gsqa, 0.0)
    c_iota = jax.lax.broadcasted_iota(jnp.int32, (C, 1), 0)
    onehot_t = c_iota == t_cls
    z = x - jnp.where(onehot_t, margin, 0.0)
    zmax = jnp.max(z, axis=0, keepdims=True)
    lse = zmax + jnp.log(jnp.sum(jnp.exp(z - zmax), axis=0, keepdims=True))
    zt = jnp.sum(jnp.where(onehot_t, z, 0.0), axis=0, keepdims=True)
    nll = lse - zt

    aw = ax1 - ax0
    ah = ay1 - ay0
    acx = ax0 + 0.5 * aw
    acy = ay0 + 0.5 * ah
    gw = gx1 - gx0
    gh = gy1 - gy0
    tstack = jnp.concatenate(
        [(gx0 + 0.5 * gw - acx) / aw,
         (gy0 + 0.5 * gh - acy) / ah,
         jnp.log(gw / aw),
         jnp.log(gh / ah)], axis=0)
    d = r - tstack
    ad = jnp.abs(d)
    sl1 = jnp.where(ad < 1.0, 0.5 * d * d, ad - 0.5)
    sl1sum = jnp.sum(sl1, axis=0, keepdims=True)

    stack4 = jnp.concatenate(
        [jnp.where(keep, nll, 0.0), keepf,
         jnp.where(is_pos, sl1sum, 0.0), posf], axis=0)
    s4 = jnp.sum(stack4, axis=1, keepdims=True)

    sums_ref[...] += s4[None]


def _per_shard(interpret, cls, reg, anc, annP, annQ):
    bl = cls.shape[0]
    clsT = jnp.transpose(cls, (0, 2, 1))
    regT = jnp.transpose(reg, (0, 2, 1))
    ancT = jnp.transpose(anc, (1, 0))
    nb = pl.cdiv(A, BLOCK_A)
    grid = (bl, nb)
    out_shape = jax.ShapeDtypeStruct((bl, 4, 128), jnp.float32)
    out_spec = pl.BlockSpec((1, 4, 128), lambda b, i: (b, 0, 0))
    return pl.pallas_call(
        _loss_kernel,
        grid=grid,
        in_specs=[
            pl.BlockSpec((1, C, BLOCK_A), lambda b, i: (b, 0, i)),
            pl.BlockSpec((1, 4, BLOCK_A), lambda b, i: (b, 0, i)),
            pl.BlockSpec((4, BLOCK_A), lambda b, i: (0, i)),
            pl.BlockSpec((1, M, 8), lambda b, i: (b, 0, 0)),
            pl.BlockSpec((1, 8, M), lambda b, i: (b, 0, 0)),
        ],
        out_specs=out_spec,
        out_shape=out_shape,
        interpret=interpret,
    )(clsT, regT, ancT, annP, annQ)


@functools.partial(jax.jit, static_argnames=("interpret",))
def kernel(classifications, regressions, anchors, annotations, interpret=False):
    bw = annotations[:, :, 2] - annotations[:, :, 0]
    bh = annotations[:, :, 3] - annotations[:, :, 1]
    area = bw * bh
    annP = jnp.stack([
        annotations[:, :, 0], annotations[:, :, 1],
        annotations[:, :, 2], annotations[:, :, 3],
        annotations[:, :, 4], jnp.sqrt(area), area,
        (annotations[:, :, 4] != -1.0).astype(jnp.float32),
    ], axis=2)
    annQ = jnp.transpose(annP, (0, 2, 1))

    ndev = max(d for d in (1, 2, 4, 8) if len(jax.devices()) >= d and B % d == 0)
    mesh = jax.sharding.Mesh(jax.devices()[:ndev], ("d",))
    P = jax.sharding.PartitionSpec
    sums = jax.shard_map(
        functools.partial(_per_shard, interpret),
        mesh=mesh,
        in_specs=(P("d"), P("d"), P(), P("d"), P("d")),
        out_specs=P("d"),
        check_vma=False,
    )(classifications, regressions, anchors[0], annP, annQ)
    s_nll = sums[:, 0, 0]
    s_keep = sums[:, 1, 0]
    s_sl1 = sums[:, 2, 0]
    s_pos = sums[:, 3, 0]
    clf = s_nll / s_keep
    reg = s_sl1 / (4.0 * s_pos)
    return clf, reg, s_pos[:, None]


# speedup vs baseline: 1.9678x; 1.9678x over previous
import functools

import jax
import jax.numpy as jnp
from jax.experimental import pallas as pl

A = 100000
B = 8
C = 20
M = 64
MARGIN = 8.0

BLOCK_A = 8192


def _loss_kernel(cls_ref, reg_ref, anc_ref, ann_ref, annq_ref, sums_ref):
    i = pl.program_id(1)

    @pl.when(i == 0)
    def _init():
        sums_ref[...] = jnp.zeros_like(sums_ref)

    x = cls_ref[0]
    r = reg_ref[0]
    ax0 = anc_ref[0:1, :]
    ay0 = anc_ref[1:2, :]
    ax1 = anc_ref[2:3, :]
    ay1 = anc_ref[3:4, :]
    q = ann_ref[0]

    bx0 = q[:, 0:1]
    by0 = q[:, 1:2]
    bx1 = q[:, 2:3]
    by1 = q[:, 3:4]
    blab = q[:, 4:5]
    bsqa = q[:, 5:6]
    barea = q[:, 6:7]
    bvalid = q[:, 7:8]

    area_a = (ax1 - ax0) * (ay1 - ay0)
    iw = jnp.maximum(jnp.minimum(ax1, bx1) - jnp.maximum(ax0, bx0), 0.0)
    ih = jnp.maximum(jnp.minimum(ay1, by1) - jnp.maximum(ay0, by0), 0.0)
    inter = iw * ih
    union = area_a + barea - inter
    iou = inter / jnp.maximum(union, 1e-8)
    iou = jnp.where(bvalid > 0.0, iou, -1.0)

    best = jnp.max(iou, axis=0, keepdims=True)
    m_iota = jax.lax.broadcasted_iota(jnp.int32, (M, 1), 0)
    argsel = jnp.min(jnp.where(iou == best, m_iota, M), axis=0, keepdims=True)
    onehot = (m_iota == argsel).astype(jnp.float32)

    qT = annq_ref[0]
    gall = jax.lax.dot_general(
        qT, onehot, (((1,), (0,)), ((), ())),
        preferred_element_type=jnp.float32)
    gx0 = gall[0:1, :]
    gy0 = gall[1:2, :]
    gx1 = gall[2:3, :]
    gy1 = gall[3:4, :]
    glab = gall[4:5, :]
    gsqa = gall[5:6, :]

    lane = jax.lax.broadcasted_iota(jnp.int32, (1, BLOCK_A), 1)
    in_range = (i * BLOCK_A + lane) < A
    is_pos = jnp.logical_and(best >= 0.5, in_range)
    keep = jnp.logical_and(jnp.logical_or(is_pos, best < 0.4), in_range)
    posf = jnp.where(is_pos, 1.0, 0.0)
    keepf = jnp.where(keep, 1.0, 0.0)

    t_cls = jnp.where(is_pos, (glab + 0.5).astype(jnp.int32), 0)
    margin = jnp.where(is_pos, MARGIN / gsqa, 0.0)
    c_iota = jax.lax.broadcasted_iota(jnp.int32, (C, 1), 0)
    onehot_t = c_iota == t_cls
    z = x - jnp.where(onehot_t, margin, 0.0)
    zmax = jnp.max(z, axis=0, keepdims=True)
    lse = zmax + jnp.log(jnp.sum(jnp.exp(z - zmax), axis=0, keepdims=True))
    zt = jnp.sum(jnp.where(onehot_t, z, 0.0), axis=0, keepdims=True)
    nll = lse - zt

    aw = ax1 - ax0
    ah = ay1 - ay0
    acx = ax0 + 0.5 * aw
    acy = ay0 + 0.5 * ah
    gw = gx1 - gx0
    gh = gy1 - gy0
    tstack = jnp.concatenate(
        [(gx0 + 0.5 * gw - acx) / aw,
         (gy0 + 0.5 * gh - acy) / ah,
         jnp.log(gw / aw),
         jnp.log(gh / ah)], axis=0)
    d = r - tstack
    ad = jnp.abs(d)
    sl1 = jnp.where(ad < 1.0, 0.5 * d * d, ad - 0.5)
    sl1sum = jnp.sum(sl1, axis=0, keepdims=True)

    stack4 = jnp.concatenate(
        [jnp.where(keep, nll, 0.0), keepf,
         jnp.where(is_pos, sl1sum, 0.0), posf], axis=0)
    s4 = jnp.sum(stack4, axis=1, keepdims=True)

    sums_ref[...] += s4[None]


def _per_shard(interpret, cls, reg, anc, annP, annQ):
    bl = cls.shape[0]
    clsT = jnp.transpose(cls, (0, 2, 1))
    regT = jnp.transpose(reg, (0, 2, 1))
    ancT = jnp.transpose(anc, (1, 0))
    nb = pl.cdiv(A, BLOCK_A)
    grid = (bl, nb)
    out_shape = jax.ShapeDtypeStruct((bl, 4, 128), jnp.float32)
    out_spec = pl.BlockSpec((1, 4, 128), lambda b, i: (b, 0, 0))
    return pl.pallas_call(
        _loss_kernel,
        grid=grid,
        in_specs=[
            pl.BlockSpec((1, C, BLOCK_A), lambda b, i: (b, 0, i)),
            pl.BlockSpec((1, 4, BLOCK_A), lambda b, i: (b, 0, i)),
            pl.BlockSpec((4, BLOCK_A), lambda b, i: (0, i)),
            pl.BlockSpec((1, M, 8), lambda b, i: (b, 0, 0)),
            pl.BlockSpec((1, 8, M), lambda b, i: (b, 0, 0)),
        ],
        out_specs=out_spec,
        out_shape=out_shape,
        interpret=interpret,
    )(clsT, regT, ancT, annP, annQ)


@functools.partial(jax.jit, static_argnames=("interpret",))
def kernel(classifications, regressions, anchors, annotations, interpret=False):
    bw = annotations[:, :, 2] - annotations[:, :, 0]
    bh = annotations[:, :, 3] - annotations[:, :, 1]
    area = bw * bh
    annP = jnp.stack([
        annotations[:, :, 0], annotations[:, :, 1],
        annotations[:, :, 2], annotations[:, :, 3],
        annotations[:, :, 4], jnp.sqrt(area), area,
        (annotations[:, :, 4] != -1.0).astype(jnp.float32),
    ], axis=2)
    annQ = jnp.transpose(annP, (0, 2, 1))

    sums = _per_shard(interpret, classifications, regressions, anchors[0],
                      annP, annQ)
    s_nll = sums[:, 0, 0]
    s_keep = sums[:, 1, 0]
    s_sl1 = sums[:, 2, 0]
    s_pos = sums[:, 3, 0]
    clf = s_nll / s_keep
    reg = s_sl1 / (4.0 * s_pos)
    return clf, reg, s_pos[:, None]


# speedup vs baseline: 2.0421x; 1.0378x over previous
import functools

import jax
import jax.numpy as jnp
from jax.experimental import pallas as pl

A = 100000
B = 8
C = 20
M = 64
MARGIN = 8.0

BLOCK_A = 10240


def _loss_kernel(cls_ref, reg_ref, anc_ref, ann_ref, annq_ref, sums_ref):
    i = pl.program_id(1)

    @pl.when(i == 0)
    def _init():
        sums_ref[...] = jnp.zeros_like(sums_ref)

    x = cls_ref[0]
    r = reg_ref[0]
    ax0 = anc_ref[0:1, :]
    ay0 = anc_ref[1:2, :]
    ax1 = anc_ref[2:3, :]
    ay1 = anc_ref[3:4, :]
    q = ann_ref[0]

    bx0 = q[:, 0:1]
    by0 = q[:, 1:2]
    bx1 = q[:, 2:3]
    by1 = q[:, 3:4]
    blab = q[:, 4:5]
    bsqa = q[:, 5:6]
    barea = q[:, 6:7]
    bvalid = q[:, 7:8]

    area_a = (ax1 - ax0) * (ay1 - ay0)
    iw = jnp.maximum(jnp.minimum(ax1, bx1) - jnp.maximum(ax0, bx0), 0.0)
    ih = jnp.maximum(jnp.minimum(ay1, by1) - jnp.maximum(ay0, by0), 0.0)
    inter = iw * ih
    union = area_a + barea - inter
    iou = inter / jnp.maximum(union, 1e-8)
    iou = jnp.where(bvalid > 0.0, iou, -1.0)

    best = jnp.max(iou, axis=0, keepdims=True)
    m_iota = jax.lax.broadcasted_iota(jnp.int32, (M, 1), 0)
    argsel = jnp.min(jnp.where(iou == best, m_iota, M), axis=0, keepdims=True)
    onehot = (m_iota == argsel).astype(jnp.float32)

    qT = annq_ref[0]
    gall = jax.lax.dot_general(
        qT, onehot, (((1,), (0,)), ((), ())),
        preferred_element_type=jnp.float32)
    gx0 = gall[0:1, :]
    gy0 = gall[1:2, :]
    gx1 = gall[2:3, :]
    gy1 = gall[3:4, :]
    glab = gall[4:5, :]
    gsqa = gall[5:6, :]

    lane = jax.lax.broadcasted_iota(jnp.int32, (1, BLOCK_A), 1)
    in_range = (i * BLOCK_A + lane) < A
    is_pos = jnp.logical_and(best >= 0.5, in_range)
    keep = jnp.logical_and(jnp.logical_or(is_pos, best < 0.4), in_range)
    posf = jnp.where(is_pos, 1.0, 0.0)
    keepf = jnp.where(keep, 1.0, 0.0)

    t_cls = jnp.where(is_pos, (glab + 0.5).astype(jnp.int32), 0)
    margin = jnp.where(is_pos, MARGIN / gsqa, 0.0)
    c_iota = jax.lax.broadcasted_iota(jnp.int32, (C, 1), 0)
    onehot_t = c_iota == t_cls
    z = x - jnp.where(onehot_t, margin, 0.0)
    zmax = jnp.max(z, axis=0, keepdims=True)
    lse = zmax + jnp.log(jnp.sum(jnp.exp(z - zmax), axis=0, keepdims=True))
    zt = jnp.sum(jnp.where(onehot_t, z, 0.0), axis=0, keepdims=True)
    nll = lse - zt

    aw = ax1 - ax0
    ah = ay1 - ay0
    acx = ax0 + 0.5 * aw
    acy = ay0 + 0.5 * ah
    gw = gx1 - gx0
    gh = gy1 - gy0
    tstack = jnp.concatenate(
        [(gx0 + 0.5 * gw - acx) / aw,
         (gy0 + 0.5 * gh - acy) / ah,
         jnp.log(gw / aw),
         jnp.log(gh / ah)], axis=0)
    d = r - tstack
    ad = jnp.abs(d)
    sl1 = jnp.where(ad < 1.0, 0.5 * d * d, ad - 0.5)
    sl1sum = jnp.sum(sl1, axis=0, keepdims=True)

    stack4 = jnp.concatenate(
        [jnp.where(keep, nll, 0.0), keepf,
         jnp.where(is_pos, sl1sum, 0.0), posf], axis=0)
    s4 = jnp.sum(stack4, axis=1, keepdims=True)

    sums_ref[...] += s4[None]


def _per_shard(interpret, cls, reg, anc, annP, annQ):
    bl = cls.shape[0]
    clsT = jnp.transpose(cls, (0, 2, 1))
    regT = jnp.transpose(reg, (0, 2, 1))
    ancT = jnp.transpose(anc, (1, 0))
    nb = pl.cdiv(A, BLOCK_A)
    grid = (bl, nb)
    out_shape = jax.ShapeDtypeStruct((bl, 4, 128), jnp.float32)
    out_spec = pl.BlockSpec((1, 4, 128), lambda b, i: (b, 0, 0))
    return pl.pallas_call(
        _loss_kernel,
        grid=grid,
        in_specs=[
            pl.BlockSpec((1, C, BLOCK_A), lambda b, i: (b, 0, i)),
            pl.BlockSpec((1, 4, BLOCK_A), lambda b, i: (b, 0, i)),
            pl.BlockSpec((4, BLOCK_A), lambda b, i: (0, i)),
            pl.BlockSpec((1, M, 8), lambda b, i: (b, 0, 0)),
            pl.BlockSpec((1, 8, M), lambda b, i: (b, 0, 0)),
        ],
        out_specs=out_spec,
        out_shape=out_shape,
        interpret=interpret,
    )(clsT, regT, ancT, annP, annQ)


@functools.partial(jax.jit, static_argnames=("interpret",))
def kernel(classifications, regressions, anchors, annotations, interpret=False):
    bw = annotations[:, :, 2] - annotations[:, :, 0]
    bh = annotations[:, :, 3] - annotations[:, :, 1]
    area = bw * bh
    annP = jnp.stack([
        annotations[:, :, 0], annotations[:, :, 1],
        annotations[:, :, 2], annotations[:, :, 3],
        annotations[:, :, 4], jnp.sqrt(area), area,
        (annotations[:, :, 4] != -1.0).astype(jnp.float32),
    ], axis=2)
    annQ = jnp.transpose(annP, (0, 2, 1))

    sums = _per_shard(interpret, classifications, regressions, anchors[0],
                      annP, annQ)
    s_nll = sums[:, 0, 0]
    s_keep = sums[:, 1, 0]
    s_sl1 = sums[:, 2, 0]
    s_pos = sums[:, 3, 0]
    clf = s_nll / s_keep
    reg = s_sl1 / (4.0 * s_pos)
    return clf, reg, s_pos[:, None]


# speedup vs baseline: 2.0452x; 1.0016x over previous
import functools

import jax
import jax.numpy as jnp
from jax.experimental import pallas as pl

A = 100000
B = 8
C = 20
M = 64
MARGIN = 8.0

BLOCK_A = 10240


def _loss_kernel(cls_ref, reg_ref, anc_ref, ann_ref, annq_ref, sums_ref):
    i = pl.program_id(1)

    @pl.when(i == 0)
    def _init():
        sums_ref[...] = jnp.zeros_like(sums_ref)

    x = cls_ref[0]
    r = reg_ref[0]
    ax0 = anc_ref[0:1, :]
    ay0 = anc_ref[1:2, :]
    ax1 = anc_ref[2:3, :]
    ay1 = anc_ref[3:4, :]
    q = ann_ref[0]

    bx0 = q[:, 0:1]
    by0 = q[:, 1:2]
    bx1 = q[:, 2:3]
    by1 = q[:, 3:4]
    blab = q[:, 4:5]
    bsqa = q[:, 5:6]
    barea = q[:, 6:7]
    bvalid = q[:, 7:8]

    area_a = (ax1 - ax0) * (ay1 - ay0)
    iw = jnp.maximum(jnp.minimum(ax1, bx1) - jnp.maximum(ax0, bx0), 0.0)
    ih = jnp.maximum(jnp.minimum(ay1, by1) - jnp.maximum(ay0, by0), 0.0)
    inter = iw * ih
    union = area_a + barea - inter
    iou = inter / jnp.maximum(union, 1e-8)
    iou = jnp.where(bvalid > 0.0, iou, -1.0)

    best = jnp.max(iou, axis=0, keepdims=True)
    m_iota = jax.lax.broadcasted_iota(jnp.int32, (M, 1), 0)
    argsel = jnp.min(jnp.where(iou == best, m_iota, M), axis=0, keepdims=True)
    onehot = (m_iota == argsel).astype(jnp.float32)

    qT = annq_ref[0]
    gall = jax.lax.dot_general(
        qT, onehot, (((1,), (0,)), ((), ())),
        preferred_element_type=jnp.float32)
    gx0 = gall[0:1, :]
    gy0 = gall[1:2, :]
    gx1 = gall[2:3, :]
    gy1 = gall[3:4, :]
    glab = gall[4:5, :]
    gsqa = gall[5:6, :]

    lane = jax.lax.broadcasted_iota(jnp.int32, (1, BLOCK_A), 1)
    in_range = (i * BLOCK_A + lane) < A
    is_pos = jnp.logical_and(best >= 0.5, in_range)
    keep = jnp.logical_and(jnp.logical_or(is_pos, best < 0.4), in_range)
    posf = jnp.where(is_pos, 1.0, 0.0)
    keepf = jnp.where(keep, 1.0, 0.0)

    t_cls = jnp.where(is_pos, (glab + 0.5).astype(jnp.int32), 0)
    margin = jnp.where(is_pos, MARGIN / gsqa, 0.0)
    c_iota = jax.lax.broadcasted_iota(jnp.int32, (C, 1), 0)
    onehot_t = c_iota == t_cls
    z = x - jnp.where(onehot_t, margin, 0.0)
    zmax = jnp.max(z, axis=0, keepdims=True)
    lse = zmax + jnp.log(jnp.sum(jnp.exp(z - zmax), axis=0, keepdims=True))
    zt = jnp.sum(jnp.where(onehot_t, z, 0.0), axis=0, keepdims=True)
    nll = lse - zt

    aw = ax1 - ax0
    ah = ay1 - ay0
    acx = ax0 + 0.5 * aw
    acy = ay0 + 0.5 * ah
    gw = gx1 - gx0
    gh = gy1 - gy0
    tstack = jnp.concatenate(
        [(gx0 + 0.5 * gw - acx) / aw,
         (gy0 + 0.5 * gh - acy) / ah,
         jnp.log(gw / aw),
         jnp.log(gh / ah)], axis=0)
    d = r - tstack
    ad = jnp.abs(d)
    sl1 = jnp.where(ad < 1.0, 0.5 * d * d, ad - 0.5)
    sl1sum = jnp.sum(sl1, axis=0, keepdims=True)

    stack4 = jnp.concatenate(
        [jnp.where(keep, nll, 0.0), keepf,
         jnp.where(is_pos, sl1sum, 0.0), posf], axis=0)
    s4 = jnp.sum(stack4, axis=1, keepdims=True)

    sums_ref[...] += s4[None]


def _per_shard(cls, reg, anc, annP, annQ):
    bl = cls.shape[0]
    clsT = jnp.transpose(cls, (0, 2, 1))
    regT = jnp.transpose(reg, (0, 2, 1))
    ancT = jnp.transpose(anc, (1, 0))
    nb = pl.cdiv(A, BLOCK_A)
    grid = (bl, nb)
    out_shape = jax.ShapeDtypeStruct((bl, 4, 128), jnp.float32)
    out_spec = pl.BlockSpec((1, 4, 128), lambda b, i: (b, 0, 0))
    return pl.pallas_call(
        _loss_kernel,
        grid=grid,
        in_specs=[
            pl.BlockSpec((1, C, BLOCK_A), lambda b, i: (b, 0, i)),
            pl.BlockSpec((1, 4, BLOCK_A), lambda b, i: (b, 0, i)),
            pl.BlockSpec((4, BLOCK_A), lambda b, i: (0, i)),
            pl.BlockSpec((1, M, 8), lambda b, i: (b, 0, 0)),
            pl.BlockSpec((1, 8, M), lambda b, i: (b, 0, 0)),
        ],
        out_specs=out_spec,
        out_shape=out_shape,
    )(clsT, regT, ancT, annP, annQ)


@jax.jit
def kernel(classifications, regressions, anchors, annotations):
    bw = annotations[:, :, 2] - annotations[:, :, 0]
    bh = annotations[:, :, 3] - annotations[:, :, 1]
    area = bw * bh
    annP = jnp.stack([
        annotations[:, :, 0], annotations[:, :, 1],
        annotations[:, :, 2], annotations[:, :, 3],
        annotations[:, :, 4], jnp.sqrt(area), area,
        (annotations[:, :, 4] != -1.0).astype(jnp.float32),
    ], axis=2)
    annQ = jnp.transpose(annP, (0, 2, 1))

    sums = _per_shard(classifications, regressions, anchors[0], annP, annQ)
    s_nll = sums[:, 0, 0]
    s_keep = sums[:, 1, 0]
    s_sl1 = sums[:, 2, 0]
    s_pos = sums[:, 3, 0]
    clf = s_nll / s_keep
    reg = s_sl1 / (4.0 * s_pos)
    return clf, reg, s_pos[:, None]
